# baseline (speedup 1.0000x reference)
import functools

import jax
import jax.numpy as jnp
from jax import lax
from jax.experimental import pallas as pl
from jax.experimental.pallas import tpu as pltpu
from jax.experimental.pallas import tpu_sc as plsc

N_CHROM = 4
CHROM = 25000
D_IN = 256
DIM = 128
EPS = 1e-5

_NC = 2
_NS = 16
_NW = _NC * _NS
_LANES = 16

_GCHUNK = 128
_BBLK = 1024
_NBUCKET = N_CHROM + 1


def _bucket(v):
    c = ((v > CHROM).astype(jnp.int32) + (v > 2 * CHROM).astype(jnp.int32)
         + (v > 3 * CHROM).astype(jnp.int32))
    return jnp.where(v >= 1, c, N_CHROM)


def _sc_count(x):
    b = x.shape[0]
    b_per_w = b // _NW
    mesh = plsc.VectorSubcoreMesh(core_axis_name="c", subcore_axis_name="s")

    @functools.partial(
        pl.kernel,
        mesh=mesh,
        compiler_params=pltpu.CompilerParams(needs_layout_passes=False),
        out_type=jax.ShapeDtypeStruct((_NW, _LANES), jnp.int32),
        scratch_types=[
            pltpu.VMEM((b_per_w,), jnp.int32),
            pltpu.VMEM((_LANES,), jnp.int32),
        ],
    )
    def k(x_hbm, out_hbm, x_v, cnt_v):
        wid = lax.axis_index("s") * _NC + lax.axis_index("c")
        base = wid * b_per_w
        pltpu.sync_copy(x_hbm.at[pl.ds(base, b_per_w)], x_v)
        lane = lax.iota(jnp.int32, _LANES)
        acc = jnp.zeros((_LANES,), jnp.int32)
        for i in range(b_per_w // _LANES):
            v = x_v[pl.ds(i * _LANES, _LANES)]
            c = _bucket(v)
            for cb in range(_NBUCKET):
                n = plsc.all_reduce_population_count(c == cb)
                acc = acc + jnp.where(lane == cb, n, 0)
        cnt_v[pl.ds(0, _LANES)] = acc
        pltpu.sync_copy(cnt_v, out_hbm.at[wid])

    return k(x)


def _sc_gather_scatter(table_flat, x, woff, npad):
    b = x.shape[0]
    b_per_w = b // _NW
    n_chunks = b_per_w // _GCHUNK
    d = table_flat.shape[1]
    mesh = plsc.VectorSubcoreMesh(core_axis_name="c", subcore_axis_name="s")

    @functools.partial(
        pl.kernel,
        mesh=mesh,
        compiler_params=pltpu.CompilerParams(needs_layout_passes=False),
        out_type=(jax.ShapeDtypeStruct((npad, d), jnp.float32),
                  jax.ShapeDtypeStruct((b,), jnp.int32)),
        scratch_types=[
            pltpu.VMEM((b_per_w,), jnp.int32),
            pltpu.VMEM((_LANES,), jnp.int32),
            pltpu.VMEM((n_chunks, _GCHUNK), jnp.int32),
            pltpu.VMEM((n_chunks, _GCHUNK), jnp.int32),
            pltpu.VMEM((_GCHUNK, d), jnp.float32),
            pltpu.SemaphoreType.DMA,
            pltpu.SemaphoreType.DMA,
        ],
    )
    def k(table_hbm, x_hbm, woff_hbm, gperm_hbm, pos_hbm,
          x_v, off_v, idx_v, dst_v, rows_v, gsem, ssem):
        wid = lax.axis_index("s") * _NC + lax.axis_index("c")
        base = wid * b_per_w
        pltpu.sync_copy(x_hbm.at[pl.ds(base, b_per_w)], x_v)
        pltpu.sync_copy(woff_hbm.at[wid], off_v)
        lane = lax.iota(jnp.int32, _LANES)
        for j in range(n_chunks):
            for i in range(_GCHUNK // _LANES):
                v = x_v[pl.ds(j * _GCHUNK + i * _LANES, _LANES)]
                idx_v[j, pl.ds(i * _LANES, _LANES)] = jnp.maximum(
                    v - 1, jnp.zeros_like(v))
                c = _bucket(v)
                offc = plsc.load_gather(off_v, [c])
                rank = jnp.zeros((_LANES,), jnp.int32)
                delta = jnp.zeros((_LANES,), jnp.int32)
                for cb in range(_NBUCKET):
                    m = c == cb
                    r = plsc.cumsum(m.astype(jnp.int32))
                    rank = jnp.where(m, r - 1, rank)
                    n = plsc.all_reduce_population_count(m)
                    delta = delta + jnp.where(lane == cb, n, 0)
                dst_v[j, pl.ds(i * _LANES, _LANES)] = offc + rank
                off_v[pl.ds(0, _LANES)] = off_v[pl.ds(0, _LANES)] + delta
            pltpu.sync_copy(dst_v.at[j],
                            pos_hbm.at[pl.ds(base + j * _GCHUNK, _GCHUNK)])
        for j in range(n_chunks):
            pltpu.async_copy(table_hbm.at[idx_v.at[j]], rows_v, gsem).wait()
            pltpu.async_copy(rows_v, gperm_hbm.at[dst_v.at[j]], ssem).wait()

    return k(table_flat, x, woff)


def _tc_body(cm_ref, vc_ref, ct_ref, g_ref, w_ref, nw_ref, nb_ref,
             y_ref, enc_scr, stat_scr):
    p = pl.program_id(0)
    i = pl.program_id(1)

    @pl.when((p == 0) & (i == 0))
    def _init():
        stat_scr[...] = jnp.zeros_like(stat_scr)

    cm = cm_ref[i]
    vc = vc_ref[i]

    @pl.when(p == 0)
    def _pass1():
        gb = g_ref[...].astype(jnp.bfloat16)
        e = jnp.dot(gb, w_ref[0], preferred_element_type=jnp.float32)
        enc_scr[pl.ds(i * _BBLK, _BBLK), :] = e
        rid = lax.broadcasted_iota(jnp.int32, (_BBLK, DIM), 0)
        em = jnp.where(rid < vc, e, 0.0)
        s1 = jnp.sum(em, axis=0, keepdims=True)
        s2 = jnp.sum(em * em, axis=0, keepdims=True)
        stat_scr[pl.ds(cm * 8, 8), 0:DIM] += s1
        stat_scr[pl.ds(cm * 8, 8), DIM:2 * DIM] += s2

    @pl.when(p == 1)
    def _pass2():
        srow = stat_scr[pl.ds(cm * 8, 8), :][0:1, :]
        cntf = jnp.maximum(ct_ref[i].astype(jnp.float32), 1.0)
        mean = srow[:, 0:DIM] / cntf
        var = srow[:, DIM:2 * DIM] / cntf - mean * mean
        rstd = lax.rsqrt(var + EPS)
        isv = cm < N_CHROM
        scale = jnp.where(isv, rstd, 0.0)
        sh = jnp.where(isv, mean * rstd, 0.0)
        enc = enc_scr[pl.ds(i * _BBLK, _BBLK), :]
        t = jnp.tanh(enc * scale - sh)
        y_ref[...] = jnp.dot(t, nw_ref[...],
                             preferred_element_type=jnp.float32) + nb_ref[...]


def _tc_forward(block_chrom, block_vcnt, block_cnt, g_perm, w_stack,
                next_wt, next_b2, nblk):
    npad = nblk * _BBLK
    grid_spec = pltpu.PrefetchScalarGridSpec(
        num_scalar_prefetch=3,
        grid=(2, nblk),
        in_specs=[
            pl.BlockSpec((_BBLK, D_IN),
                         lambda p, i, cm, vc, ct: (i * (1 - p), 0)),
            pl.BlockSpec((1, D_IN, DIM),
                         lambda p, i, cm, vc, ct: (cm[i], 0, 0)),
            pl.BlockSpec((DIM, DIM), lambda p, i, cm, vc, ct: (0, 0)),
            pl.BlockSpec((1, DIM), lambda p, i, cm, vc, ct: (0, 0)),
        ],
        out_specs=pl.BlockSpec((_BBLK, DIM), lambda p, i, cm, vc, ct: (i, 0)),
        scratch_shapes=[
            pltpu.VMEM((npad, DIM), jnp.float32),
            pltpu.VMEM((_NBUCKET * 8, 2 * DIM), jnp.float32),
        ],
    )
    return pl.pallas_call(
        _tc_body,
        grid_spec=grid_spec,
        out_shape=jax.ShapeDtypeStruct((npad, DIM), jnp.float32),
    )(block_chrom, block_vcnt, block_cnt, g_perm, w_stack, next_wt, next_b2)


def _sc_unpermute(y_perm, pos):
    b = pos.shape[0]
    b_per_w = b // _NW
    n_chunks = b_per_w // _GCHUNK
    mesh = plsc.VectorSubcoreMesh(core_axis_name="c", subcore_axis_name="s")

    @functools.partial(
        pl.kernel,
        mesh=mesh,
        out_type=jax.ShapeDtypeStruct((b, DIM), jnp.float32),
        scratch_types=[
            pltpu.VMEM((b_per_w,), jnp.int32),
            pltpu.VMEM((_GCHUNK, DIM), jnp.float32),
            pltpu.VMEM((_GCHUNK, DIM), jnp.float32),
            pltpu.SemaphoreType.DMA,
            pltpu.SemaphoreType.DMA,
        ],
    )
    def k(y_hbm, pos_hbm, out_hbm, pos_v, rows0_v, rows1_v, sem0, sem1):
        wid = lax.axis_index("s") * _NC + lax.axis_index("c")
        base = wid * b_per_w
        pltpu.sync_copy(pos_hbm.at[pl.ds(base, b_per_w)], pos_v)
        rows = (rows0_v, rows1_v)
        sems = (sem0, sem1)
        copies = [None, None]
        for j in range(n_chunks):
            s = j % 2
            if copies[s] is not None:
                copies[s].wait()
                pltpu.sync_copy(
                    rows[s], out_hbm.at[pl.ds(base + (j - 2) * _GCHUNK,
                                              _GCHUNK)])
            cp = pltpu.async_copy(
                y_hbm.at[pos_v.at[pl.ds(j * _GCHUNK, _GCHUNK)]],
                rows[s], sems[s])
            copies[s] = cp
        for j in range(n_chunks - 2, n_chunks):
            s = j % 2
            copies[s].wait()
            pltpu.sync_copy(
                rows[s], out_hbm.at[pl.ds(base + j * _GCHUNK, _GCHUNK)])

    return k(y_perm, pos)


def kernel(x, tables, Ws, next_W, next_b):
    b = x.shape[0]
    nblk = b // _BBLK + _NBUCKET
    npad = nblk * _BBLK
    table_flat = tables.reshape(N_CHROM * CHROM, D_IN)

    counts = _sc_count(x)
    totals = jnp.sum(counts, axis=0)[:_NBUCKET]
    blocks_c = (totals + _BBLK - 1) // _BBLK
    cumb = jnp.cumsum(blocks_c)
    seg_start = (cumb - blocks_c) * _BBLK
    karr = jnp.arange(nblk, dtype=jnp.int32)
    block_chrom = jnp.minimum(
        jnp.sum((karr[:, None] >= cumb[None, :]).astype(jnp.int32), axis=1),
        N_CHROM).astype(jnp.int32)
    bc_start = jnp.take(seg_start, block_chrom)
    bc_total = jnp.take(totals, block_chrom)
    block_vcnt = jnp.clip(bc_start + bc_total - karr * _BBLK,
                          0, _BBLK).astype(jnp.int32)
    exc = jnp.cumsum(counts, axis=0) - counts
    woff = exc[:, :_NBUCKET] + seg_start[None, :]
    woff16 = jnp.pad(woff, ((0, 0), (0, _LANES - _NBUCKET)))

    g_perm, pos = _sc_gather_scatter(table_flat, x, woff16, npad)

    w_stack = jnp.concatenate(
        [jnp.transpose(Ws, (0, 2, 1)),
         jnp.zeros((1, D_IN, DIM), jnp.float32)], axis=0
    ).astype(jnp.bfloat16)
    block_cnt = bc_total.astype(jnp.int32)
    next_wt = next_W.T
    next_b2 = next_b.reshape(1, DIM)

    y_perm = _tc_forward(block_chrom, block_vcnt, block_cnt, g_perm, w_stack,
                         next_wt, next_b2, nblk)
    return _sc_unpermute(y_perm, pos)


# speedup vs baseline: 1.3182x; 1.3182x over previous
import functools

import jax
import jax.numpy as jnp
from jax import lax
from jax.experimental import pallas as pl
from jax.experimental.pallas import tpu as pltpu
from jax.experimental.pallas import tpu_sc as plsc

N_CHROM = 4
CHROM = 25000
D_IN = 256
DIM = 128
EPS = 1e-5

_NC = 2
_NS = 16
_NW = _NC * _NS
_LANES = 16

_GCHUNK = 128


def _sc_gather(table_flat, x):
    b = x.shape[0]
    b_per_w = b // _NW
    n_chunks = b_per_w // _GCHUNK
    d = table_flat.shape[1]
    mesh = plsc.VectorSubcoreMesh(core_axis_name="c", subcore_axis_name="s")

    @functools.partial(
        pl.kernel,
        mesh=mesh,
        out_type=jax.ShapeDtypeStruct((b, d), jnp.float32),
        scratch_types=[
            pltpu.VMEM((b_per_w,), jnp.int32),
            pltpu.VMEM((n_chunks, _GCHUNK), jnp.int32),
            pltpu.VMEM((_GCHUNK, d), jnp.float32),
            pltpu.VMEM((_GCHUNK, d), jnp.float32),
            pltpu.SemaphoreType.DMA,
            pltpu.SemaphoreType.DMA,
        ],
    )
    def k(table_hbm, x_hbm, out_hbm, x_v, idx_v, rows0_v, rows1_v, sem0, sem1):
        wid = lax.axis_index("s") * _NC + lax.axis_index("c")
        base = wid * b_per_w
        pltpu.sync_copy(x_hbm.at[pl.ds(base, b_per_w)], x_v)
        for j in range(n_chunks):
            for i in range(_GCHUNK // _LANES):
                v = x_v[pl.ds(j * _GCHUNK + i * _LANES, _LANES)]
                idx_v[j, pl.ds(i * _LANES, _LANES)] = jnp.maximum(
                    v - 1, jnp.zeros_like(v)
                )
        rows = (rows0_v, rows1_v)
        sems = (sem0, sem1)
        copies = [None, None]
        for j in range(n_chunks):
            s = j % 2
            if copies[s] is not None:
                copies[s].wait()
                pltpu.sync_copy(
                    rows[s], out_hbm.at[pl.ds(base + (j - 2) * _GCHUNK, _GCHUNK)]
                )
            cp = pltpu.async_copy(table_hbm.at[idx_v.at[j]], rows[s], sems[s])
            copies[s] = cp
        for j in range(n_chunks - 2, n_chunks):
            s = j % 2
            copies[s].wait()
            pltpu.sync_copy(
                rows[s], out_hbm.at[pl.ds(base + j * _GCHUNK, _GCHUNK)]
            )

    return k(table_flat, x)


def _tc_body(x_ref, xr_ref, g_ref, w_ref, nw_ref, nb_ref, out_ref,
             enc_scr, stat_scr, *, bblk):
    p = pl.program_id(0)
    b = pl.program_id(1)

    @pl.when((p == 0) & (b == 0))
    def _init():
        stat_scr[...] = jnp.zeros_like(stat_scr)

    xb = x_ref[...]
    m01 = xb <= 2 * CHROM
    m0 = xb <= CHROM
    m2 = xb <= 3 * CHROM

    @pl.when(p == 0)
    def _pass1():
        xr = xr_ref[...]
        bounds = [(i * CHROM + 1, (i + 1) * CHROM) for i in range(N_CHROM)]
        row_masks = [(xr >= lo) & (xr <= hi) for lo, hi in bounds]
        mf_t = jnp.concatenate(
            [m.astype(jnp.float32) for m in row_masks], axis=0)
        gb = g_ref[...].astype(jnp.bfloat16)
        e_all = jnp.dot(gb, w_ref[...],
                        preferred_element_type=jnp.float32)
        e0 = e_all[:, 0:DIM]
        e1 = e_all[:, DIM:2 * DIM]
        e2 = e_all[:, 2 * DIM:3 * DIM]
        e3 = e_all[:, 3 * DIM:4 * DIM]
        esel = jnp.where(m01, jnp.where(m0, e0, e1),
                         jnp.where(m2, e2, e3))
        stat_scr[:, 0:DIM] += jnp.dot(mf_t, esel,
                                      preferred_element_type=jnp.float32)
        stat_scr[:, DIM:2 * DIM] += jnp.dot(
            mf_t, esel * esel, preferred_element_type=jnp.float32)
        for i in range(N_CHROM):
            ci = jnp.sum(mf_t[i:i + 1, :])
            stat_scr[pl.ds(i, 1), pl.ds(2 * DIM, DIM)] = (
                stat_scr[pl.ds(i, 1), pl.ds(2 * DIM, DIM)] + ci)
        enc_scr[pl.ds(b * bblk, bblk), :] = esel

    @pl.when(p == 1)
    def _pass2():
        stats = stat_scr[...]
        cnt = jnp.maximum(stats[:, 2 * DIM:3 * DIM], 1.0)
        mean = stats[:, 0:DIM] / cnt
        var = stats[:, DIM:2 * DIM] / cnt - mean * mean
        rstd = lax.rsqrt(var + EPS)
        shift = mean * rstd
        valid = xb >= 1
        scale = jnp.where(
            m01,
            jnp.where(m0, rstd[0:1, :], rstd[1:2, :]),
            jnp.where(m2, rstd[2:3, :], rstd[3:4, :]))
        offs = jnp.where(
            m01,
            jnp.where(m0, shift[0:1, :], shift[1:2, :]),
            jnp.where(m2, shift[2:3, :], shift[3:4, :]))
        scale = jnp.where(valid, scale, 0.0)
        offs = jnp.where(valid, offs, 0.0)
        enc = enc_scr[pl.ds(b * bblk, bblk), :]
        normalized = enc * scale - offs
        t = jnp.tanh(normalized)
        out_ref[...] = jnp.dot(t, nw_ref[...],
                               preferred_element_type=jnp.float32) + nb_ref[...]


def _tc_forward(g, x2, xr, w_cat, next_wt, next_b2):
    b = g.shape[0]
    bblk = 2048
    nb = b // bblk
    grid = (2, nb)
    return pl.pallas_call(
        functools.partial(_tc_body, bblk=bblk),
        grid=grid,
        in_specs=[
            pl.BlockSpec((bblk, 1), lambda p, i: (i, 0)),
            pl.BlockSpec((1, bblk), lambda p, i: (0, i)),
            pl.BlockSpec((bblk, D_IN), lambda p, i: (i * (1 - p), 0)),
            pl.BlockSpec((D_IN, N_CHROM * DIM), lambda p, i: (0, 0)),
            pl.BlockSpec((DIM, DIM), lambda p, i: (0, 0)),
            pl.BlockSpec((1, DIM), lambda p, i: (0, 0)),
        ],
        out_specs=pl.BlockSpec((bblk, DIM), lambda p, i: (i, 0)),
        out_shape=jax.ShapeDtypeStruct((b, DIM), jnp.float32),
        scratch_shapes=[
            pltpu.VMEM((b, DIM), jnp.float32),
            pltpu.VMEM((N_CHROM, 3 * DIM), jnp.float32),
        ],
    )(x2, xr, g, w_cat, next_wt, next_b2)


def kernel(x, tables, Ws, next_W, next_b):
    b = x.shape[0]
    table_flat = tables.reshape(N_CHROM * CHROM, D_IN)
    g = _sc_gather(table_flat, x)
    x2 = x.reshape(b, 1)
    xr = x.reshape(1, b)
    w_cat = jnp.transpose(Ws, (2, 0, 1)).reshape(
        D_IN, N_CHROM * DIM).astype(jnp.bfloat16)
    next_wt = next_W.T
    next_b2 = next_b.reshape(1, DIM)
    return _tc_forward(g, x2, xr, w_cat, next_wt, next_b2)


# speedup vs baseline: 1.3540x; 1.0272x over previous
import functools

import jax
import jax.numpy as jnp
from jax import lax
from jax.experimental import pallas as pl
from jax.experimental.pallas import tpu as pltpu
from jax.experimental.pallas import tpu_sc as plsc

N_CHROM = 4
CHROM = 25000
D_IN = 256
DIM = 128
EPS = 1e-5

_NC = 2
_NS = 16
_NW = _NC * _NS
_LANES = 16

_GCHUNK = 128


def _sc_gather(table_flat, x):
    b = x.shape[0]
    b_per_w = b // _NW
    n_chunks = b_per_w // _GCHUNK
    d = table_flat.shape[1]
    mesh = plsc.VectorSubcoreMesh(core_axis_name="c", subcore_axis_name="s")

    @functools.partial(
        pl.kernel,
        mesh=mesh,
        out_type=jax.ShapeDtypeStruct((b, d), jnp.float32),
        scratch_types=[
            pltpu.VMEM((b_per_w,), jnp.int32),
            pltpu.VMEM((n_chunks, _GCHUNK), jnp.int32),
            pltpu.VMEM((_GCHUNK, d), jnp.float32),
            pltpu.VMEM((_GCHUNK, d), jnp.float32),
            pltpu.SemaphoreType.DMA,
            pltpu.SemaphoreType.DMA,
        ],
    )
    def k(table_hbm, x_hbm, out_hbm, x_v, idx_v, rows0_v, rows1_v, sem0, sem1):
        wid = lax.axis_index("s") * _NC + lax.axis_index("c")
        base = wid * b_per_w
        pltpu.sync_copy(x_hbm.at[pl.ds(base, b_per_w)], x_v)
        for j in range(n_chunks):
            for i in range(_GCHUNK // _LANES):
                v = x_v[pl.ds(j * _GCHUNK + i * _LANES, _LANES)]
                idx_v[j, pl.ds(i * _LANES, _LANES)] = jnp.maximum(
                    v - 1, jnp.zeros_like(v)
                )
        rows = (rows0_v, rows1_v)
        sems = (sem0, sem1)
        copies = [None, None]
        for j in range(n_chunks):
            s = j % 2
            if copies[s] is not None:
                copies[s].wait()
                pltpu.sync_copy(
                    rows[s], out_hbm.at[pl.ds(base + (j - 2) * _GCHUNK, _GCHUNK)]
                )
            cp = pltpu.async_copy(table_hbm.at[idx_v.at[j]], rows[s], sems[s])
            copies[s] = cp
        for j in range(n_chunks - 2, n_chunks):
            s = j % 2
            copies[s].wait()
            pltpu.sync_copy(
                rows[s], out_hbm.at[pl.ds(base + j * _GCHUNK, _GCHUNK)]
            )

    return k(table_flat, x)


def _tc_body(x_ref, xr_ref, g_ref, w_ref, nw_ref, nb_ref, out_ref,
             enc_scr, stat_scr, *, bblk):
    p = pl.program_id(0)
    b = pl.program_id(1)

    @pl.when((p == 0) & (b == 0))
    def _init():
        stat_scr[...] = jnp.zeros_like(stat_scr)

    xb = x_ref[...]
    m01 = xb <= 2 * CHROM
    m0 = xb <= CHROM
    m2 = xb <= 3 * CHROM

    @pl.when(p == 0)
    def _pass1():
        xr = xr_ref[...]
        bounds = [(i * CHROM + 1, (i + 1) * CHROM) for i in range(N_CHROM)]
        row_masks = [(xr >= lo) & (xr <= hi) for lo, hi in bounds]
        mf_t = jnp.concatenate(
            [m.astype(jnp.float32) for m in row_masks], axis=0)
        gb = g_ref[...].astype(jnp.bfloat16)
        e_all = jnp.dot(gb, w_ref[...],
                        preferred_element_type=jnp.float32)
        e0 = e_all[:, 0:DIM]
        e1 = e_all[:, DIM:2 * DIM]
        e2 = e_all[:, 2 * DIM:3 * DIM]
        e3 = e_all[:, 3 * DIM:4 * DIM]
        esel = jnp.where(m01, jnp.where(m0, e0, e1),
                         jnp.where(m2, e2, e3))
        stat_scr[:, 0:DIM] += jnp.dot(mf_t, esel,
                                      preferred_element_type=jnp.float32)
        stat_scr[:, DIM:2 * DIM] += jnp.dot(
            mf_t, esel * esel, preferred_element_type=jnp.float32)
        for i in range(N_CHROM):
            ci = jnp.sum(mf_t[i:i + 1, :])
            stat_scr[pl.ds(i, 1), pl.ds(2 * DIM, DIM)] = (
                stat_scr[pl.ds(i, 1), pl.ds(2 * DIM, DIM)] + ci)
        enc_scr[pl.ds(b * bblk, bblk), :] = esel

    @pl.when(p == 1)
    def _pass2():
        stats = stat_scr[...]
        cnt = jnp.maximum(stats[:, 2 * DIM:3 * DIM], 1.0)
        mean = stats[:, 0:DIM] / cnt
        var = stats[:, DIM:2 * DIM] / cnt - mean * mean
        rstd = lax.rsqrt(var + EPS)
        shift = mean * rstd
        valid = xb >= 1
        scale = jnp.where(
            m01,
            jnp.where(m0, rstd[0:1, :], rstd[1:2, :]),
            jnp.where(m2, rstd[2:3, :], rstd[3:4, :]))
        offs = jnp.where(
            m01,
            jnp.where(m0, shift[0:1, :], shift[1:2, :]),
            jnp.where(m2, shift[2:3, :], shift[3:4, :]))
        scale = jnp.where(valid, scale, 0.0)
        offs = jnp.where(valid, offs, 0.0)
        enc = enc_scr[pl.ds(b * bblk, bblk), :]
        normalized = enc * scale - offs
        t = jnp.tanh(normalized)
        out_ref[...] = jnp.dot(t, nw_ref[...],
                               preferred_element_type=jnp.float32) + nb_ref[...]


def _tc_forward(g, x2, xr, w_cat, next_wt, next_b2):
    b = g.shape[0]
    bblk = 4096
    nb = b // bblk
    grid = (2, nb)
    return pl.pallas_call(
        functools.partial(_tc_body, bblk=bblk),
        grid=grid,
        in_specs=[
            pl.BlockSpec((bblk, 1), lambda p, i: (i, 0)),
            pl.BlockSpec((1, bblk), lambda p, i: (0, i)),
            pl.BlockSpec((bblk, D_IN), lambda p, i: (i * (1 - p), 0)),
            pl.BlockSpec((D_IN, N_CHROM * DIM), lambda p, i: (0, 0)),
            pl.BlockSpec((DIM, DIM), lambda p, i: (0, 0)),
            pl.BlockSpec((1, DIM), lambda p, i: (0, 0)),
        ],
        out_specs=pl.BlockSpec((bblk, DIM), lambda p, i: (i, 0)),
        out_shape=jax.ShapeDtypeStruct((b, DIM), jnp.float32),
        scratch_shapes=[
            pltpu.VMEM((b, DIM), jnp.float32),
            pltpu.VMEM((N_CHROM, 3 * DIM), jnp.float32),
        ],
    )(x2, xr, g, w_cat, next_wt, next_b2)


def kernel(x, tables, Ws, next_W, next_b):
    b = x.shape[0]
    table_flat = tables.reshape(N_CHROM * CHROM, D_IN)
    g = _sc_gather(table_flat, x)
    x2 = x.reshape(b, 1)
    xr = x.reshape(1, b)
    w_cat = jnp.transpose(Ws, (2, 0, 1)).reshape(
        D_IN, N_CHROM * DIM).astype(jnp.bfloat16)
    next_wt = next_W.T
    next_b2 = next_b.reshape(1, DIM)
    return _tc_forward(g, x2, xr, w_cat, next_wt, next_b2)
